# baseline (speedup 1.0000x reference)
import functools

import jax
import jax.numpy as jnp
from jax import lax
from jax.experimental import pallas as pl
from jax.experimental.pallas import tpu as pltpu
from jax.experimental.pallas import tpu_sc as plsc

_MARGIN = 1.0
_MAX_ITER = 15
_NEAREST = 3
_NUM = 2
_BIG = 1e30

_B = 256
_C = 128
_NSUB = 16
_L = 16
_HALF = 128


def _prep_body(f_ref, c_ref, packed_ref, same_ref):
    f = f_ref[:]
    c = c_ref[:]
    B, K = f.shape
    C = c.shape[0]

    hi = lax.Precision.HIGHEST
    ones_row = jnp.ones((1, K), jnp.float32)
    cn_col = lax.dot_general(
        c * c, ones_row, (((1,), (1,)), ((), ())),
        precision=hi, preferred_element_type=jnp.float32)
    fn_row = lax.dot_general(
        ones_row, f * f, (((1,), (1,)), ((), ())),
        precision=hi, preferred_element_type=jnp.float32)
    cf = lax.dot_general(
        c, f, (((1,), (1,)), ((), ())),
        precision=hi, preferred_element_type=jnp.float32)
    Dt = jnp.sqrt(jnp.maximum(cn_col + fn_row - 2.0 * cf, 0.0))

    cn_row = lax.dot_general(
        ones_row, c * c, (((1,), (1,)), ((), ())),
        precision=hi, preferred_element_type=jnp.float32)
    cc = lax.dot_general(
        c, c, (((1,), (1,)), ((), ())),
        precision=hi, preferred_element_type=jnp.float32)
    S2 = jnp.maximum(cn_col + cn_row - 2.0 * cc, 0.0)
    rowc = lax.broadcasted_iota(jnp.int32, (C, C), 0)
    colc = lax.broadcasted_iota(jnp.int32, (C, C), 1)
    S2 = jnp.where(rowc == colc, 0.0, S2)

    near = jnp.zeros((C, C), jnp.float32)
    work = S2
    for _ in range(_NEAREST):
        m = jnp.min(work, axis=1, keepdims=True)
        eq = work == m
        first = jnp.min(jnp.where(eq, colc, C + 1), axis=1, keepdims=True)
        oh = colc == first
        near = jnp.where(oh, 1.0, near)
        work = jnp.where(oh, _BIG, work)
    trust = 1.0 - near

    rowt = lax.broadcasted_iota(jnp.int32, (C, B), 0)
    colt = lax.broadcasted_iota(jnp.int32, (C, B), 1)
    lblt = colt // _NUM
    onehot = (lblt == rowt).astype(jnp.float32)
    tmask_t = lax.dot_general(
        trust, onehot, (((1,), (0,)), ((), ())),
        preferred_element_type=jnp.float32)

    own = rowt == lblt
    same_ref[:] = jnp.sum(jnp.where(own, Dt, 0.0), axis=0,
                          keepdims=True)
    signed = jnp.where(tmask_t > 0.5, -(Dt + 1.0), Dt)
    packed_ref[:] = jnp.where(own, _BIG, signed)


@functools.partial(
    pl.kernel,
    mesh=plsc.VectorSubcoreMesh(
        core_axis_name="c", subcore_axis_name="s", num_cores=1),
    out_type=[
        jax.ShapeDtypeStruct((_NSUB, _L), jnp.float32),
        jax.ShapeDtypeStruct((_L,), jnp.float32),
    ],
    scratch_types=[
        pltpu.VMEM((_C, _HALF), jnp.float32),
        pltpu.VMEM((1, _HALF), jnp.float32),
        pltpu.VMEM((_L,), jnp.float32),
        pltpu.VMEM((_NSUB, _L), jnp.float32),
    ],
)
def _sc_select(packed_hbm, same_hbm, part_hbm, out_hbm,
               pk_v, same_v, acc_v, sh_v):
    sid = lax.axis_index("s")
    half = sid // 8
    off = pl.multiple_of((sid % 8) * _L, _L)
    hbase = pl.multiple_of(half * _HALF, _HALF)
    pltpu.sync_copy(packed_hbm.at[:, pl.ds(hbase, _HALF)], pk_v)
    pltpu.sync_copy(same_hbm.at[:, pl.ds(hbase, _HALF)], same_v)

    def tdist(cx):
        v = pk_v[cx, pl.ds(off, _L)]
        return jnp.where(v < -0.5, -1.0 - v, _BIG)

    umins = [tdist(cx) for cx in range(4)]
    for cx in range(4, _C):
        umins[cx % 4] = jnp.minimum(umins[cx % 4], tdist(cx))
    u = jnp.minimum(jnp.minimum(umins[0], umins[1]),
                    jnp.minimum(umins[2], umins[3]))

    def closer(cx):
        v = pk_v[cx, pl.ds(off, _L)]
        d = jnp.where(v < -0.5, -1.0 - v, v)
        return jnp.where(d < u, 1.0, 0.0)

    cnts = [closer(cx) for cx in range(4)]
    for cx in range(4, _C):
        cnts[cx % 4] = cnts[cx % 4] + closer(cx)
    cnt = (cnts[0] + cnts[1]) + (cnts[2] + cnts[3])

    md = jnp.where(
        cnt < float(_MAX_ITER), jnp.where(u < _BIG * 0.5, u, 0.0), 0.0)
    hinge = jnp.maximum(
        _MARGIN + same_v[0, pl.ds(off, _L)] - md, 0.0)
    acc_v[:] = hinge
    pltpu.sync_copy(acc_v, part_hbm.at[sid])
    plsc.subcore_barrier()

    @pl.when(sid == 0)
    def _():
        pltpu.sync_copy(part_hbm, sh_v)
        tot = sh_v[0, :]
        for w in range(1, _NSUB):
            tot = tot + sh_v[w, :]
        lane = lax.broadcasted_iota(jnp.int32, (_L,), 0)
        for s in (8, 4, 2, 1):
            tot = tot + tot.at[lane ^ s].get(mode="promise_in_bounds")
        acc_v[:] = tot * (1.0 / _B)
        pltpu.sync_copy(acc_v, out_hbm)


def kernel(feature, centers):
    packed, same = pl.pallas_call(
        _prep_body,
        out_shape=[
            jax.ShapeDtypeStruct((_C, _B), jnp.float32),
            jax.ShapeDtypeStruct((1, _B), jnp.float32),
        ],
    )(feature, centers)
    _, loss = _sc_select(packed, same)
    return loss[0]


# speedup vs baseline: 1.0804x; 1.0804x over previous
import functools

import jax
import jax.numpy as jnp
from jax import lax
from jax.experimental import pallas as pl
from jax.experimental.pallas import tpu as pltpu
from jax.experimental.pallas import tpu_sc as plsc

_MARGIN = 1.0
_MAX_ITER = 15
_NEAREST = 3
_NUM = 2
_BIG = 1e30

_B = 256
_C = 128
_NCORES = 2
_NSUB = 16
_NW = _NCORES * _NSUB
_ROWS = _B // _NW
_RPW = _B // _NSUB
_L = 16
_NCH = _C // _L


def _prep_body(f_ref, c_ref, workd_ref, tworkd_ref, same_ref):
    f = f_ref[:]
    c = c_ref[:]
    B, K = f.shape
    C = c.shape[0]

    hi = lax.Precision.HIGHEST
    fn = jnp.sum(f * f, axis=1, keepdims=True)
    ones_row = jnp.ones((1, K), jnp.float32)
    cn_row = lax.dot_general(
        ones_row, c * c, (((1,), (1,)), ((), ())),
        precision=hi, preferred_element_type=jnp.float32)
    fc = lax.dot_general(
        f, c, (((1,), (1,)), ((), ())),
        precision=hi, preferred_element_type=jnp.float32)
    D = jnp.sqrt(jnp.maximum(fn + cn_row - 2.0 * fc, 0.0))

    cc = lax.dot_general(
        c, c, (((1,), (1,)), ((), ())),
        precision=hi, preferred_element_type=jnp.float32)
    cn_col = lax.dot_general(
        c * c, ones_row, (((1,), (1,)), ((), ())),
        precision=hi, preferred_element_type=jnp.float32)
    S2 = jnp.maximum(cn_col + cn_row - 2.0 * cc, 0.0)
    rowc = lax.broadcasted_iota(jnp.int32, (C, C), 0)
    colc = lax.broadcasted_iota(jnp.int32, (C, C), 1)
    S2 = jnp.where(rowc == colc, 0.0, S2)

    near = jnp.zeros((C, C), jnp.float32)
    work = S2
    for _ in range(_NEAREST):
        m = jnp.min(work, axis=1, keepdims=True)
        eq = work == m
        first = jnp.min(jnp.where(eq, colc, C + 1), axis=1, keepdims=True)
        oh = colc == first
        near = jnp.where(oh, 1.0, near)
        work = jnp.where(oh, _BIG, work)
    trust = 1.0 - near

    rowb = lax.broadcasted_iota(jnp.int32, (B, C), 0)
    colb = lax.broadcasted_iota(jnp.int32, (B, C), 1)
    lbl = rowb // _NUM
    onehot = (colb == lbl).astype(jnp.float32)
    tmask = lax.dot_general(
        onehot, trust, (((1,), (1,)), ((), ())),
        preferred_element_type=jnp.float32)

    own = colb == lbl
    same = jnp.sum(jnp.where(own, D, 0.0), axis=1, keepdims=True)
    same_ref[:] = jnp.broadcast_to(same, (B, _L))
    workd = jnp.where(own, _BIG, D)
    workd_ref[:] = workd
    tworkd_ref[:] = jnp.where(tmask > 0.5, workd, _BIG)


@functools.partial(
    pl.kernel,
    mesh=plsc.VectorSubcoreMesh(
        core_axis_name="c", subcore_axis_name="s", num_cores=1),
    out_type=[
        jax.ShapeDtypeStruct((_NSUB, _L), jnp.float32),
        jax.ShapeDtypeStruct((_L,), jnp.float32),
    ],
    scratch_types=[
        pltpu.VMEM((_RPW, _C), jnp.float32),
        pltpu.VMEM((_RPW, _C), jnp.float32),
        pltpu.VMEM((_RPW, _L), jnp.float32),
        pltpu.VMEM((_L,), jnp.float32),
        pltpu.VMEM((_NSUB, _L), jnp.float32),
        pltpu.SemaphoreType.DMA,
        pltpu.SemaphoreType.DMA,
        pltpu.SemaphoreType.DMA,
    ],
)
def _sc_select(workd_hbm, tworkd_hbm, same_hbm, part_hbm, out_hbm,
               wd_v, td_v, same_v, acc_v, sh_v, sem1, sem2, sem3):
    sid = lax.axis_index("s")
    base = sid * _RPW
    cp1 = pltpu.async_copy(workd_hbm.at[pl.ds(base, _RPW)], wd_v, sem1)
    cp2 = pltpu.async_copy(tworkd_hbm.at[pl.ds(base, _RPW)], td_v, sem2)
    cp3 = pltpu.async_copy(same_hbm.at[pl.ds(base, _RPW)], same_v, sem3)
    cp1.wait()
    cp2.wait()
    cp3.wait()

    lane = lax.broadcasted_iota(jnp.int32, (_L,), 0)

    def splat_min(v):
        for s in (8, 4, 2, 1):
            v = jnp.minimum(v, v.at[lane ^ s].get(mode="promise_in_bounds"))
        return v

    def splat_sum(v):
        for s in (8, 4, 2, 1):
            v = v + v.at[lane ^ s].get(mode="promise_in_bounds")
        return v

    hacc = jnp.zeros((_L,), jnp.float32)
    for r in range(_RPW):
        tm = td_v[r, pl.ds(0, _L)]
        for k in range(1, _NCH):
            tm = jnp.minimum(tm, td_v[r, pl.ds(k * _L, _L)])
        uv = splat_min(tm)
        cnt = jnp.zeros((_L,), jnp.float32)
        for k in range(_NCH):
            dk = wd_v[r, pl.ds(k * _L, _L)]
            cnt = cnt + jnp.where(dk < uv, 1.0, 0.0)
        cnt = splat_sum(cnt)
        md = jnp.where(
            cnt < float(_MAX_ITER), jnp.where(uv < _BIG * 0.5, uv, 0.0), 0.0)
        hacc = hacc + jnp.maximum(_MARGIN + same_v[r, :] - md, 0.0)
    acc_v[:] = hacc
    pltpu.sync_copy(acc_v, part_hbm.at[sid])
    plsc.subcore_barrier()

    @pl.when(sid == 0)
    def _():
        pltpu.sync_copy(part_hbm, sh_v)
        tot = sh_v[0, :]
        for w in range(1, _NSUB):
            tot = tot + sh_v[w, :]
        acc_v[:] = tot * (1.0 / _B)
        pltpu.sync_copy(acc_v, out_hbm)


def kernel(feature, centers):
    workd, tworkd, same = pl.pallas_call(
        _prep_body,
        out_shape=[
            jax.ShapeDtypeStruct((_B, _C), jnp.float32),
            jax.ShapeDtypeStruct((_B, _C), jnp.float32),
            jax.ShapeDtypeStruct((_B, _L), jnp.float32),
        ],
    )(feature, centers)
    _, loss = _sc_select(workd, tworkd, same)
    return loss[0]
